# Initial kernel scaffold; baseline (speedup 1.0000x reference)
import jax
import jax.numpy as jnp
from jax.experimental import pallas as pl


def kernel(x, batch_index, rbatch_index, W1, b1, W2, b2):
    raise NotImplementedError("write your pallas kernel here")


# speedup vs baseline: 2.6010x; 2.6010x over previous
import functools

import jax
import jax.numpy as jnp
from jax import lax
from jax.experimental import pallas as pl
from jax.experimental.pallas import tpu as pltpu
from jax.experimental.pallas import tpu_sc as plsc

N = 32768
D = 128
B = 16
BLK = 2048
NB = N // BLK

SC_NC = 2
SC_NS = 16
NW = SC_NC * SC_NS
B_PER_W = N // NW
CHUNK = 512


def _pool_body(x_ref, idx_ref, w1t_ref, b1_ref, w2t_ref, b2_ref, invc_ref,
               out_ref, acc_sum, acc_max):
    i = pl.program_id(0)

    @pl.when(i == 0)
    def _init():
        acc_sum[...] = jnp.zeros_like(acc_sum)
        acc_max[...] = jnp.full_like(acc_max, -3e38)

    idxv = idx_ref[0, 0, :]
    h = jnp.dot(x_ref[...], w1t_ref[...],
                preferred_element_type=jnp.float32) + b1_ref[...]

    segT = lax.broadcasted_iota(jnp.int32, (B, BLK), 0)
    ohT = (segT == idxv[None, :]).astype(jnp.float32)
    acc_sum[...] += jnp.dot(ohT, h, preferred_element_type=jnp.float32)

    s_lo = jnp.min(idxv)
    s_hi = jnp.max(idxv)
    idxm = jnp.broadcast_to(jnp.reshape(idxv, (BLK, 1)), (BLK, D))
    rows = lax.broadcasted_iota(jnp.int32, (B, 1), 0)

    def seg_step(s, carry):
        @pl.when((s >= s_lo) & (s <= s_hi))
        def _():
            colmax = jnp.max(jnp.where(idxm == s, h, -3e38), axis=0,
                             keepdims=True)
            upd = jnp.maximum(acc_max[...], colmax)
            acc_max[...] = jnp.where(rows == s, upd, acc_max[...])
        return carry

    lax.fori_loop(0, B, seg_step, 0)

    @pl.when(i == NB - 1)
    def _finish():
        pooled = jnp.concatenate(
            [acc_sum[...] * invc_ref[...], acc_max[...]], axis=1)
        out_ref[...] = jnp.dot(pooled, w2t_ref[...],
                               preferred_element_type=jnp.float32) + b2_ref[...]


def _pool_tc(x, idx3, w1t, b1r, w2t, b2r, invc):
    return pl.pallas_call(
        _pool_body,
        grid=(NB,),
        in_specs=[
            pl.BlockSpec((BLK, D), lambda i: (i, 0)),
            pl.BlockSpec((1, 1, BLK), lambda i: (i, 0, 0)),
            pl.BlockSpec((D, D), lambda i: (0, 0)),
            pl.BlockSpec((1, D), lambda i: (0, 0)),
            pl.BlockSpec((2 * D, D), lambda i: (0, 0)),
            pl.BlockSpec((1, D), lambda i: (0, 0)),
            pl.BlockSpec((B, D), lambda i: (0, 0)),
        ],
        out_specs=pl.BlockSpec((B, D), lambda i: (0, 0)),
        out_shape=jax.ShapeDtypeStruct((B, D), jnp.float32),
        scratch_shapes=[
            pltpu.VMEM((B, D), jnp.float32),
            pltpu.VMEM((B, D), jnp.float32),
        ],
    )(x, idx3, w1t, b1r, w2t, b2r, invc)


def _bcast_body(table_hbm, idx_hbm, out_hbm, idx_v, rows_v, sem):
    wid = lax.axis_index("s") * SC_NC + lax.axis_index("c")
    base = wid * B_PER_W
    pltpu.sync_copy(idx_hbm.at[pl.ds(base, B_PER_W)], idx_v)
    for c in range(B_PER_W // CHUNK):
        pltpu.async_copy(
            table_hbm.at[idx_v.at[pl.ds(c * CHUNK, CHUNK)]], rows_v, sem
        ).wait()
        pltpu.sync_copy(rows_v, out_hbm.at[pl.ds(base + c * CHUNK, CHUNK)])


@functools.cache
def _bcast_sc():
    return pl.kernel(
        _bcast_body,
        out_type=jax.ShapeDtypeStruct((N, D), jnp.float32),
        mesh=plsc.VectorSubcoreMesh(core_axis_name="c", subcore_axis_name="s"),
        scratch_types=[
            pltpu.VMEM((B_PER_W,), jnp.int32),
            pltpu.VMEM((CHUNK, D), jnp.float32),
            pltpu.SemaphoreType.DMA,
        ],
    )


def kernel(x, batch_index, rbatch_index, W1, b1, W2, b2):
    idx = batch_index.astype(jnp.int32)
    counts = (rbatch_index[1:] - rbatch_index[:-1]).astype(jnp.float32)
    invc = jnp.broadcast_to(
        (1.0 / jnp.maximum(counts, 1.0)).reshape(B, 1), (B, D))
    pooled = _pool_tc(
        x, idx.reshape(NB, 1, BLK), W1.T, b1.reshape(1, D), W2.T,
        b2.reshape(1, D), invc)
    return _bcast_sc()(pooled, idx)


# speedup vs baseline: 6.8770x; 2.6440x over previous
import functools

import jax
import jax.numpy as jnp
from jax import lax
from jax.experimental import pallas as pl
from jax.experimental.pallas import tpu as pltpu
from jax.experimental.pallas import tpu_sc as plsc

N = 32768
D = 128
B = 16
BLK = 2048
NB = N // BLK

SC_NC = 2
SC_NS = 16
NW = SC_NC * SC_NS
B_PER_W = N // NW


def _pool_body(x_ref, idx_ref, w1t_ref, b1_ref, w2t_ref, b2_ref, invc_ref,
               out_ref, acc_sum, acc_max):
    i = pl.program_id(0)

    @pl.when(i == 0)
    def _init():
        acc_sum[...] = jnp.zeros_like(acc_sum)
        acc_max[...] = jnp.full_like(acc_max, -3e38)

    idxv = idx_ref[0, 0, :]
    h = jnp.dot(x_ref[...], w1t_ref[...],
                preferred_element_type=jnp.float32) + b1_ref[...]

    segT = lax.broadcasted_iota(jnp.int32, (B, BLK), 0)
    ohT = (segT == idxv[None, :]).astype(jnp.float32)
    acc_sum[...] += jnp.dot(ohT, h, preferred_element_type=jnp.float32)

    s_lo = jnp.min(idxv)
    s_hi = jnp.max(idxv)
    idxm = jnp.broadcast_to(jnp.reshape(idxv, (BLK, 1)), (BLK, D))
    rows = lax.broadcasted_iota(jnp.int32, (B, 1), 0)

    def seg_step(s, carry):
        @pl.when((s >= s_lo) & (s <= s_hi))
        def _():
            colmax = jnp.max(jnp.where(idxm == s, h, -3e38), axis=0,
                             keepdims=True)
            upd = jnp.maximum(acc_max[...], colmax)
            acc_max[...] = jnp.where(rows == s, upd, acc_max[...])
        return carry

    lax.fori_loop(0, B, seg_step, 0)

    @pl.when(i == NB - 1)
    def _finish():
        pooled = jnp.concatenate(
            [acc_sum[...] * invc_ref[...], acc_max[...]], axis=1)
        out_ref[...] = jnp.dot(pooled, w2t_ref[...],
                               preferred_element_type=jnp.float32) + b2_ref[...]


def _pool_tc(x, idx3, w1t, b1r, w2t, b2r, invc):
    return pl.pallas_call(
        _pool_body,
        grid=(NB,),
        in_specs=[
            pl.BlockSpec((BLK, D), lambda i: (i, 0)),
            pl.BlockSpec((1, 1, BLK), lambda i: (i, 0, 0)),
            pl.BlockSpec((D, D), lambda i: (0, 0)),
            pl.BlockSpec((1, D), lambda i: (0, 0)),
            pl.BlockSpec((2 * D, D), lambda i: (0, 0)),
            pl.BlockSpec((1, D), lambda i: (0, 0)),
            pl.BlockSpec((B, D), lambda i: (0, 0)),
        ],
        out_specs=pl.BlockSpec((B, D), lambda i: (0, 0)),
        out_shape=jax.ShapeDtypeStruct((B, D), jnp.float32),
        scratch_shapes=[
            pltpu.VMEM((B, D), jnp.float32),
            pltpu.VMEM((B, D), jnp.float32),
        ],
    )(x, idx3, w1t, b1r, w2t, b2r, invc)


REP = 128


def _bcast_body(table_hbm, rb_hbm, out_hbm, table_v, rb_v, rep_v):
    wid = lax.axis_index("s") * SC_NC + lax.axis_index("c")
    base = wid * B_PER_W
    lim = base + B_PER_W

    pltpu.sync_copy(table_hbm, table_v)
    pltpu.sync_copy(rb_hbm, rb_v)

    rb_lo = rb_v[pl.ds(0, 16)]
    rb_hi = rb_v[pl.ds(16, 16)]
    ends = [rb_lo[k] for k in range(16)]
    ends.append(rb_hi[0])

    for s in range(B):
        lo = jnp.maximum(ends[s], base)
        hi = jnp.minimum(ends[s + 1], lim)

        @pl.when(hi > lo)
        def _run(s=s, lo=lo, hi=hi):
            row = [table_v[pl.ds(s * D + c * 16, 16)] for c in range(8)]
            nrows = jnp.minimum(hi - lo, REP)

            def fill(j, carry):
                for c in range(8):
                    rep_v[pl.ds(j * D + c * 16, 16)] = row[c]
                return carry

            lax.fori_loop(0, nrows, fill, 0)

            span = hi - lo
            nfull = span // REP

            def blast(j, carry):
                pltpu.sync_copy(
                    rep_v,
                    out_hbm.at[pl.ds((lo + j * REP) * D, REP * D)])
                return carry

            lax.fori_loop(0, nfull, blast, 0)

            sz = REP // 2
            while sz >= 1:
                rem_off = lo + (span & ~(2 * sz - 1))

                @pl.when((span & sz) != 0)
                def _tail(sz=sz, rem_off=rem_off):
                    pltpu.sync_copy(
                        rep_v.at[pl.ds(0, sz * D)],
                        out_hbm.at[pl.ds(rem_off * D, sz * D)])

                sz //= 2


@functools.cache
def _bcast_sc():
    return pl.kernel(
        _bcast_body,
        out_type=jax.ShapeDtypeStruct((N * D,), jnp.float32),
        mesh=plsc.VectorSubcoreMesh(core_axis_name="c", subcore_axis_name="s"),
        scratch_types=[
            pltpu.VMEM((B * D,), jnp.float32),
            pltpu.VMEM((32,), jnp.int32),
            pltpu.VMEM((REP * D,), jnp.float32),
        ],
    )


def kernel(x, batch_index, rbatch_index, W1, b1, W2, b2):
    idx = batch_index.astype(jnp.int32)
    rb = rbatch_index.astype(jnp.int32)
    counts = (rb[1:] - rb[:-1]).astype(jnp.float32)
    invc = jnp.broadcast_to(
        (1.0 / jnp.maximum(counts, 1.0)).reshape(B, 1), (B, D))
    pooled = _pool_tc(
        x, idx.reshape(NB, 1, BLK), W1.T, b1.reshape(1, D), W2.T,
        b2.reshape(1, D), invc)
    rb32 = jnp.concatenate([rb, jnp.full((15,), N, jnp.int32)])
    return _bcast_sc()(pooled.reshape(-1), rb32).reshape(N, D)


# speedup vs baseline: 7.3540x; 1.0694x over previous
import functools

import jax
import jax.numpy as jnp
from jax import lax
from jax.experimental import pallas as pl
from jax.experimental.pallas import tpu as pltpu
from jax.experimental.pallas import tpu_sc as plsc

N = 32768
D = 128
B = 16
BLK = 2048
NB = N // BLK

SC_NC = 2
SC_NS = 16
NW = SC_NC * SC_NS
B_PER_W = N // NW


_DN_T = (((1,), (1,)), ((), ()))


def _pool_body(x_ref, idx_ref, w1_ref, b1_ref, w2_ref, b2_ref, rb_ref,
               out_ref, acc_sum, acc_max):
    i = pl.program_id(0)

    @pl.when(i == 0)
    def _init():
        acc_sum[...] = jnp.zeros_like(acc_sum)
        acc_max[...] = jnp.full_like(acc_max, -3e38)

    idxv = idx_ref[0, 0, :]
    h = lax.dot_general(x_ref[...], w1_ref[...], _DN_T,
                        preferred_element_type=jnp.float32) + b1_ref[...]

    segT = lax.broadcasted_iota(jnp.int32, (B, BLK), 0)
    ohT = (segT == idxv[None, :]).astype(jnp.float32)
    acc_sum[...] += jnp.dot(ohT, h, preferred_element_type=jnp.float32)

    s_lo = jnp.min(idxv)
    s_hi = jnp.max(idxv)
    idxm = jnp.broadcast_to(jnp.reshape(idxv, (BLK, 1)), (BLK, D))
    rows = lax.broadcasted_iota(jnp.int32, (B, 1), 0)

    def seg_step(s, carry):
        @pl.when((s >= s_lo) & (s <= s_hi))
        def _():
            colmax = jnp.max(jnp.where(idxm == s, h, -3e38), axis=0,
                             keepdims=True)
            upd = jnp.maximum(acc_max[...], colmax)
            acc_max[...] = jnp.where(rows == s, upd, acc_max[...])
        return carry

    lax.fori_loop(0, B, seg_step, 0)

    @pl.when(i == NB - 1)
    def _finish():
        cnt = (rb_ref[0, pl.ds(1, B)] - rb_ref[0, pl.ds(0, B)]).astype(
            jnp.float32)
        invc = 1.0 / jnp.maximum(cnt, 1.0)
        r_io = lax.broadcasted_iota(jnp.int32, (B, B), 0)
        c_io = lax.broadcasted_iota(jnp.int32, (B, B), 1)
        diag = jnp.where(r_io == c_io, invc[None, :], 0.0)
        mean = jnp.dot(diag, acc_sum[...],
                       preferred_element_type=jnp.float32)
        pooled = jnp.concatenate([mean, acc_max[...]], axis=1)
        out_ref[...] = lax.dot_general(
            pooled, w2_ref[...], _DN_T,
            preferred_element_type=jnp.float32) + b2_ref[...]


def _pool_tc(x, idx3, w1, b1r, w2, b2r, rb2):
    return pl.pallas_call(
        _pool_body,
        grid=(NB,),
        in_specs=[
            pl.BlockSpec((BLK, D), lambda i: (i, 0)),
            pl.BlockSpec((1, 1, BLK), lambda i: (i, 0, 0)),
            pl.BlockSpec((D, D), lambda i: (0, 0)),
            pl.BlockSpec((1, D), lambda i: (0, 0)),
            pl.BlockSpec((D, 2 * D), lambda i: (0, 0)),
            pl.BlockSpec((1, D), lambda i: (0, 0)),
            pl.BlockSpec((1, 32), lambda i: (0, 0)),
        ],
        out_specs=pl.BlockSpec((B, D), lambda i: (0, 0)),
        out_shape=jax.ShapeDtypeStruct((B, D), jnp.float32),
        scratch_shapes=[
            pltpu.VMEM((B, D), jnp.float32),
            pltpu.VMEM((B, D), jnp.float32),
        ],
    )(x, idx3, w1, b1r, w2, b2r, rb2)


REP = 128


def _bcast_body(table_hbm, rb_hbm, out_hbm, table_v, rb_v, rep_v):
    wid = lax.axis_index("s") * SC_NC + lax.axis_index("c")
    base = wid * B_PER_W
    lim = base + B_PER_W

    pltpu.sync_copy(table_hbm, table_v)
    pltpu.sync_copy(rb_hbm, rb_v)

    rb_lo = rb_v[pl.ds(0, 16)]
    rb_hi = rb_v[pl.ds(16, 16)]
    ends = [rb_lo[k] for k in range(16)]
    ends.append(rb_hi[0])

    for s in range(B):
        lo = jnp.maximum(ends[s], base)
        hi = jnp.minimum(ends[s + 1], lim)

        @pl.when(hi > lo)
        def _run(s=s, lo=lo, hi=hi):
            row = [table_v[pl.ds(s * D + c * 16, 16)] for c in range(8)]
            nrows = jnp.minimum(hi - lo, REP)

            def fill(j, carry):
                for c in range(8):
                    rep_v[pl.ds(j * D + c * 16, 16)] = row[c]
                return carry

            lax.fori_loop(0, nrows, fill, 0)

            span = hi - lo
            nfull = span // REP

            def blast(j, carry):
                pltpu.sync_copy(
                    rep_v,
                    out_hbm.at[pl.ds((lo + j * REP) * D, REP * D)])
                return carry

            lax.fori_loop(0, nfull, blast, 0)

            sz = REP // 2
            while sz >= 1:
                rem_off = lo + (span & ~(2 * sz - 1))

                @pl.when((span & sz) != 0)
                def _tail(sz=sz, rem_off=rem_off):
                    pltpu.sync_copy(
                        rep_v.at[pl.ds(0, sz * D)],
                        out_hbm.at[pl.ds(rem_off * D, sz * D)])

                sz //= 2


@functools.cache
def _bcast_sc():
    return pl.kernel(
        _bcast_body,
        out_type=jax.ShapeDtypeStruct((N * D,), jnp.float32),
        mesh=plsc.VectorSubcoreMesh(core_axis_name="c", subcore_axis_name="s"),
        scratch_types=[
            pltpu.VMEM((B * D,), jnp.float32),
            pltpu.VMEM((32,), jnp.int32),
            pltpu.VMEM((REP * D,), jnp.float32),
        ],
    )


def kernel(x, batch_index, rbatch_index, W1, b1, W2, b2):
    idx = batch_index.astype(jnp.int32)
    rb = rbatch_index.astype(jnp.int32)
    rb32 = jnp.concatenate([rb, jnp.full((15,), N, jnp.int32)])
    pooled = _pool_tc(
        x, idx.reshape(NB, 1, BLK), W1, b1.reshape(1, D), W2,
        b2.reshape(1, D), rb32.reshape(1, 32))
    return _bcast_sc()(pooled.reshape(-1), rb32).reshape(N, D)


# speedup vs baseline: 7.5462x; 1.0261x over previous
import functools

import jax
import jax.numpy as jnp
from jax import lax
from jax.experimental import pallas as pl
from jax.experimental.pallas import tpu as pltpu
from jax.experimental.pallas import tpu_sc as plsc

N = 32768
D = 128
B = 16
BLK = 2048
NB = N // BLK

SC_NC = 2
SC_NS = 16
NW = SC_NC * SC_NS
B_PER_W = N // NW


_DN_T = (((1,), (1,)), ((), ()))


def _pool_body(x_ref, idx_ref, w1_ref, b1_ref, w2_ref, b2_ref, rb_ref,
               out_ref, acc_sum, acc_max):
    i = pl.program_id(0)

    @pl.when(i == 0)
    def _init():
        acc_sum[...] = jnp.zeros_like(acc_sum)
        acc_max[...] = jnp.full_like(acc_max, -3e38)

    idxv = idx_ref[0, 0, :]
    h = lax.dot_general(x_ref[...], w1_ref[...], _DN_T,
                        preferred_element_type=jnp.float32) + b1_ref[...]

    segT = lax.broadcasted_iota(jnp.int32, (B, BLK), 0)
    ohT = (segT == idxv[None, :]).astype(jnp.float32)
    acc_sum[...] += jnp.dot(ohT, h, preferred_element_type=jnp.float32)

    s_lo = jnp.min(idxv)
    s_hi = jnp.max(idxv)
    idxm = jnp.broadcast_to(jnp.reshape(idxv, (BLK, 1)), (BLK, D))
    rows = lax.broadcasted_iota(jnp.int32, (B, 1), 0)

    for s in range(B):
        @pl.when((s >= s_lo) & (s <= s_hi))
        def _seg(s=s):
            colmax = jnp.max(jnp.where(idxm == s, h, -3e38), axis=0,
                             keepdims=True)
            upd = jnp.maximum(acc_max[...], colmax)
            acc_max[...] = jnp.where(rows == s, upd, acc_max[...])

    @pl.when(i == NB - 1)
    def _finish():
        cnt = (rb_ref[0, pl.ds(1, B)] - rb_ref[0, pl.ds(0, B)]).astype(
            jnp.float32)
        invc = 1.0 / jnp.maximum(cnt, 1.0)
        r_io = lax.broadcasted_iota(jnp.int32, (B, B), 0)
        c_io = lax.broadcasted_iota(jnp.int32, (B, B), 1)
        diag = jnp.where(r_io == c_io, invc[None, :], 0.0)
        mean = jnp.dot(diag, acc_sum[...],
                       preferred_element_type=jnp.float32)
        pooled = jnp.concatenate([mean, acc_max[...]], axis=1)
        out_ref[...] = lax.dot_general(
            pooled, w2_ref[...], _DN_T,
            preferred_element_type=jnp.float32) + b2_ref[...]


def _pool_tc(x, idx3, w1, b1r, w2, b2r, rb2):
    return pl.pallas_call(
        _pool_body,
        grid=(NB,),
        in_specs=[
            pl.BlockSpec((BLK, D), lambda i: (i, 0)),
            pl.BlockSpec((1, 1, BLK), lambda i: (i, 0, 0)),
            pl.BlockSpec((D, D), lambda i: (0, 0)),
            pl.BlockSpec((1, D), lambda i: (0, 0)),
            pl.BlockSpec((D, 2 * D), lambda i: (0, 0)),
            pl.BlockSpec((1, D), lambda i: (0, 0)),
            pl.BlockSpec((1, 32), lambda i: (0, 0)),
        ],
        out_specs=pl.BlockSpec((B, D), lambda i: (0, 0)),
        out_shape=jax.ShapeDtypeStruct((B, D), jnp.float32),
        scratch_shapes=[
            pltpu.VMEM((B, D), jnp.float32),
            pltpu.VMEM((B, D), jnp.float32),
        ],
    )(x, idx3, w1, b1r, w2, b2r, rb2)


REP = 128


def _bcast_body(table_hbm, rb_hbm, out_hbm, table_v, rb_v, rep_v):
    wid = lax.axis_index("s") * SC_NC + lax.axis_index("c")
    base = wid * B_PER_W
    lim = base + B_PER_W

    pltpu.sync_copy(table_hbm, table_v)
    pltpu.sync_copy(rb_hbm, rb_v)

    rb_lo = rb_v[pl.ds(0, 16)]
    rb_hi = rb_v[pl.ds(16, 16)]
    ends = [rb_lo[k] for k in range(16)]
    ends.append(rb_hi[0])

    for s in range(B):
        lo = jnp.maximum(ends[s], base)
        hi = jnp.minimum(ends[s + 1], lim)

        @pl.when(hi > lo)
        def _run(s=s, lo=lo, hi=hi):
            row = [table_v[pl.ds(s * D + c * 16, 16)] for c in range(8)]
            nrows = jnp.minimum(hi - lo, REP)

            def fill(j, carry):
                for c in range(8):
                    rep_v[pl.ds(j * D + c * 16, 16)] = row[c]
                return carry

            lax.fori_loop(0, nrows, fill, 0)

            span = hi - lo
            nfull = span // REP

            def blast(j, carry):
                pltpu.sync_copy(
                    rep_v,
                    out_hbm.at[pl.ds((lo + j * REP) * D, REP * D)])
                return carry

            lax.fori_loop(0, nfull, blast, 0)

            sz = REP // 2
            while sz >= 1:
                rem_off = lo + (span & ~(2 * sz - 1))

                @pl.when((span & sz) != 0)
                def _tail(sz=sz, rem_off=rem_off):
                    pltpu.sync_copy(
                        rep_v.at[pl.ds(0, sz * D)],
                        out_hbm.at[pl.ds(rem_off * D, sz * D)])

                sz //= 2


@functools.cache
def _bcast_sc():
    return pl.kernel(
        _bcast_body,
        out_type=jax.ShapeDtypeStruct((N * D,), jnp.float32),
        mesh=plsc.VectorSubcoreMesh(core_axis_name="c", subcore_axis_name="s"),
        scratch_types=[
            pltpu.VMEM((B * D,), jnp.float32),
            pltpu.VMEM((32,), jnp.int32),
            pltpu.VMEM((REP * D,), jnp.float32),
        ],
    )


def kernel(x, batch_index, rbatch_index, W1, b1, W2, b2):
    idx = batch_index.astype(jnp.int32)
    rb = rbatch_index.astype(jnp.int32)
    rb32 = jnp.concatenate([rb, jnp.full((15,), N, jnp.int32)])
    pooled = _pool_tc(
        x, idx.reshape(NB, 1, BLK), W1, b1.reshape(1, D), W2,
        b2.reshape(1, D), rb32.reshape(1, 32))
    return _bcast_sc()(pooled.reshape(-1), rb32).reshape(N, D)
